# Initial kernel scaffold; baseline (speedup 1.0000x reference)
import jax
import jax.numpy as jnp
from jax.experimental import pallas as pl


def kernel(x, c1_l1w, c1_l1b, c1_qkvw, c1_aw, c1_l2w, c1_l2b, ln_g, ln_b, c2_l1w, c2_l1b, c2_qkvw, c2_aw, c2_l2w, c2_l2b):
    raise NotImplementedError("write your pallas kernel here")


# speedup vs baseline: 3.7779x; 3.7779x over previous
import numpy as np
import jax
import jax.numpy as jnp
from jax.experimental import pallas as pl
from jax.experimental.pallas import tpu as pltpu

_H = 8
_A = 32
_DH = 16
_JP = 24
_F = 16

_SRC = [0, 0, 0, 1, 1, 2, 2, 3, 4, 4, 5, 5, 6, 7, 7, 8, 8, 8, 8, 9, 9, 10,
        11, 11, 12, 12, 13, 14, 14, 15, 15, 16]
_DST = [1, 4, 7, 0, 2, 1, 3, 2, 0, 5, 4, 6, 5, 0, 8, 7, 9, 11, 14, 8, 10, 9,
        8, 12, 11, 13, 12, 8, 15, 14, 16, 15]


def _edge_mask() -> np.ndarray:
    adj = np.zeros((17, 17), dtype=np.float32)
    adj[np.array(_SRC), np.array(_DST)] = 1.0
    a = adj + adj @ adj
    np.fill_diagonal(a, 0.0)
    src, dst = np.nonzero(a)
    src, dst = list(src), list(dst)
    for g in (17, 18):
        for i in range(17):
            src.append(i); dst.append(g)
            src.append(g); dst.append(i)
    madd = np.full((_JP, _JP), -1e30, dtype=np.float32)
    madd[np.array(src), np.array(dst)] = 0.0
    return madd


_MADD = _edge_mask()

_PERM = np.concatenate([
    np.array([h * 96 + a for h in range(_H) for a in range(_A)]),
    np.array([h * 96 + _A + a for h in range(_H) for a in range(_A)]),
    np.array([h * 96 + 2 * _A + d for h in range(_H) for d in range(_DH)]),
    np.array([h * 96 + 2 * _A + _DH + d for h in range(_H) for d in range(_DH)]),
])

_E8 = np.zeros((_H, _H * _DH), dtype=np.float32)
for _h in range(_H):
    _E8[_h, _h * _DH:(_h + 1) * _DH] = 1.0

_AW_ROWS = np.arange(_H * _A)
_AW_COLS = _AW_ROWS // _A


def _gconv(h, f, madd, e8, w1, b1, wqkv, aw_blk, w2, b2):
    h1 = jnp.dot(h, w1, preferred_element_type=jnp.float32) + b1
    qkv = jnp.dot(h1, wqkv, preferred_element_type=jnp.float32)
    q = qkv[:, 0:256].reshape(f, _JP, 256)
    k = qkv[:, 256:512].reshape(f, _JP, 256)
    v1 = qkv[:, 512:640]
    v2 = qkv[:, 640:768].reshape(f, _JP, 128)

    m = jnp.full((f, 1, _H), -1e30, dtype=jnp.float32)
    sig = jnp.zeros((f, _JP, _H), dtype=jnp.float32)
    acc = jnp.zeros((f, _JP, 128), dtype=jnp.float32)
    for j in range(19):
        kj = k[:, j:j + 1, :]
        sp = jax.nn.softplus(q + kj)
        sj = jnp.dot(sp.reshape(f * _JP, 256), aw_blk,
                     preferred_element_type=jnp.float32).reshape(f, _JP, _H)
        sj = sj + madd[:, j:j + 1].reshape(1, _JP, 1)
        mj = jnp.max(sj, axis=1, keepdims=True)
        mn = jnp.maximum(m, mj)
        sc = jnp.exp(m - mn)
        zj = jnp.exp(sj - mn)
        sc128 = jnp.dot(sc.reshape(f, _H), e8,
                        preferred_element_type=jnp.float32).reshape(f, 1, 128)
        z128 = jnp.dot(zj.reshape(f * _JP, _H), e8,
                       preferred_element_type=jnp.float32).reshape(f, _JP, 128)
        sig = sig * sc + zj
        acc = acc * sc128 + z128 * v2[:, j:j + 1, :]
        m = mn
    den = jnp.dot((sig + 1e-10).reshape(f * _JP, _H), e8,
                  preferred_element_type=jnp.float32)
    out = v1 + acc.reshape(f * _JP, 128) / den
    return jnp.dot(out, w2, preferred_element_type=jnp.float32) + b2


def _fused(x_ref, madd_ref, e8_ref,
           w1a_ref, b1a_ref, qkva_ref, awa_ref, w2a_ref, b2a_ref,
           lng_ref, lnb_ref,
           w1b_ref, b1b_ref, qkvb_ref, awb_ref, w2b_ref, b2b_ref,
           o_ref):
    f = x_ref.shape[0]
    xt = x_ref[...]
    xm = jnp.mean(xt, axis=1, keepdims=True)
    h = jnp.concatenate(
        [xt, jnp.zeros((f, 1, 256), jnp.float32), xm,
         jnp.zeros((f, _JP - 19, 256), jnp.float32)], axis=1)
    h = h.reshape(f * _JP, 256)
    madd = madd_ref[...]
    e8 = e8_ref[...]

    h = _gconv(h, f, madd, e8, w1a_ref[...], b1a_ref[...], qkva_ref[...],
               awa_ref[...], w2a_ref[...], b2a_ref[...])

    mu = jnp.mean(h, axis=-1, keepdims=True)
    var = jnp.mean((h - mu) ** 2, axis=-1, keepdims=True)
    h = (h - mu) * jax.lax.rsqrt(var + 1e-5) * lng_ref[...] + lnb_ref[...]

    h = _gconv(h, f, madd, e8, w1b_ref[...], b1b_ref[...], qkvb_ref[...],
               awb_ref[...], w2b_ref[...], b2b_ref[...])

    o_ref[...] = h.reshape(f, _JP, 256)[:, :17, :]


def kernel(x, c1_l1w, c1_l1b, c1_qkvw, c1_aw, c1_l2w, c1_l2b, ln_g, ln_b,
           c2_l1w, c2_l1b, c2_qkvw, c2_aw, c2_l2w, c2_l2b):
    B, T, J, C = x.shape
    n = B * T
    xf = x.reshape(n, J, C)

    def pack(qkvw, aw):
        wp = qkvw.T[:, _PERM]
        aw_blk = jnp.zeros((_H * _A, _H), dtype=x.dtype)
        aw_blk = aw_blk.at[_AW_ROWS, _AW_COLS].set(jnp.tile(aw[0], _H))
        return wp, aw_blk

    wp1, awb1 = pack(c1_qkvw, c1_aw)
    wp2, awb2 = pack(c2_qkvw, c2_aw)
    operands = (
        xf, jnp.asarray(_MADD), jnp.asarray(_E8),
        c1_l1w.T, c1_l1b[None], wp1, awb1, c1_l2w.T, c1_l2b[None],
        ln_g[None], ln_b[None],
        c2_l1w.T, c2_l1b[None], wp2, awb2, c2_l2w.T, c2_l2b[None],
    )

    def full_spec(a):
        return pl.BlockSpec(a.shape, lambda i, _r=len(a.shape): (0,) * _r)

    in_specs = [pl.BlockSpec((_F, J, C), lambda i: (i, 0, 0))]
    in_specs += [full_spec(a) for a in operands[1:]]

    out = pl.pallas_call(
        _fused,
        grid=(n // _F,),
        in_specs=in_specs,
        out_specs=pl.BlockSpec((_F, J, C), lambda i: (i, 0, 0)),
        out_shape=jax.ShapeDtypeStruct((n, J, C), x.dtype),
        compiler_params=pltpu.CompilerParams(
            dimension_semantics=("parallel",)),
    )(*operands)
    return out.reshape(B, T, J, C)


# speedup vs baseline: 4.9772x; 1.3175x over previous
import numpy as np
import jax
import jax.numpy as jnp
from jax.experimental import pallas as pl
from jax.experimental.pallas import tpu as pltpu

_H = 8
_A = 32
_DH = 16
_J = 19
_F = 16

_SRC = [0, 0, 0, 1, 1, 2, 2, 3, 4, 4, 5, 5, 6, 7, 7, 8, 8, 8, 8, 9, 9, 10,
        11, 11, 12, 12, 13, 14, 14, 15, 15, 16]
_DST = [1, 4, 7, 0, 2, 1, 3, 2, 0, 5, 4, 6, 5, 0, 8, 7, 9, 11, 14, 8, 10, 9,
        8, 12, 11, 13, 12, 8, 15, 14, 16, 15]


def _edge_mask() -> np.ndarray:
    adj = np.zeros((17, 17), dtype=np.float32)
    adj[np.array(_SRC), np.array(_DST)] = 1.0
    a = adj + adj @ adj
    np.fill_diagonal(a, 0.0)
    src, dst = np.nonzero(a)
    src, dst = list(src), list(dst)
    for g in (17, 18):
        for i in range(17):
            src.append(i); dst.append(g)
            src.append(g); dst.append(i)
    madd = np.full((_J, _J), -1e30, dtype=np.float32)
    madd[np.array(src), np.array(dst)] = 0.0
    return madd


_MADD = _edge_mask()

_PERM = np.concatenate([
    np.array([h * 96 + a for h in range(_H) for a in range(_A)]),
    np.array([h * 96 + _A + a for h in range(_H) for a in range(_A)]),
    np.array([h * 96 + 2 * _A + d for h in range(_H) for d in range(_DH)]),
    np.array([h * 96 + 2 * _A + _DH + d for h in range(_H) for d in range(_DH)]),
])

_E8 = np.zeros((_H, _H * _DH), dtype=np.float32)
for _h in range(_H):
    _E8[_h, _h * _DH:(_h + 1) * _DH] = 1.0

_AW_ONEHOT = (np.arange(_H)[None, :] == (np.arange(_H * _A) // _A)[:, None]
              ).astype(np.float32)


def _gconv(h, f, madd, e8, w1, b1, wqkv, aw_blk, w2, b2):
    h1 = jnp.dot(h, w1, preferred_element_type=jnp.float32) + b1
    qkv = jnp.dot(h1, wqkv, preferred_element_type=jnp.float32)
    q = qkv[:, 0:256].reshape(_J, f, 256)
    k = qkv[:, 256:512].reshape(_J, f, 256)
    v1 = qkv[:, 512:640]
    v2 = qkv[:, 640:768].reshape(_J, f, 128)

    scores = []
    m = jnp.full((f, _H), -1e30, dtype=jnp.float32)
    for j in range(_J):
        sp = jax.nn.softplus(q + k[j][None])
        sj = jnp.dot(sp.reshape(_J * f, 256), aw_blk,
                     preferred_element_type=jnp.float32).reshape(_J, f, _H)
        sj = sj + madd[:, j:j + 1].reshape(_J, 1, 1)
        scores.append(jnp.dot(sj.reshape(_J * f, _H), e8,
                              preferred_element_type=jnp.float32
                              ).reshape(_J, f, 128))
        m = jnp.maximum(m, jnp.max(sj, axis=0))
    m128 = jnp.dot(m, e8, preferred_element_type=jnp.float32)
    den = jnp.zeros((_J, f, 128), dtype=jnp.float32)
    acc = jnp.zeros((_J, f, 128), dtype=jnp.float32)
    for j in range(_J):
        zj = jnp.exp(scores[j] - m128[None])
        den = den + zj
        acc = acc + zj * v2[j][None]
    out = v1 + acc.reshape(_J * f, 128) / (den.reshape(_J * f, 128) + 1e-10)
    return jnp.dot(out, w2, preferred_element_type=jnp.float32) + b2


def _fused(x_ref, madd_ref, e8_ref,
           w1a_ref, b1a_ref, qkva_ref, awa_ref, w2a_ref, b2a_ref,
           lng_ref, lnb_ref,
           w1b_ref, b1b_ref, qkvb_ref, awb_ref, w2b_ref, b2b_ref,
           o_ref):
    f = x_ref.shape[0]
    xt = jnp.swapaxes(x_ref[...], 0, 1)
    xm = jnp.mean(xt, axis=0, keepdims=True)
    h = jnp.concatenate(
        [xt, jnp.zeros((1, f, 256), jnp.float32), xm], axis=0)
    h = h.reshape(_J * f, 256)
    madd = madd_ref[...]
    e8 = e8_ref[...]

    h = _gconv(h, f, madd, e8, w1a_ref[...], b1a_ref[...], qkva_ref[...],
               awa_ref[...], w2a_ref[...], b2a_ref[...])

    mu = jnp.mean(h, axis=-1, keepdims=True)
    var = jnp.mean((h - mu) ** 2, axis=-1, keepdims=True)
    h = (h - mu) * jax.lax.rsqrt(var + 1e-5) * lng_ref[...] + lnb_ref[...]

    h = _gconv(h, f, madd, e8, w1b_ref[...], b1b_ref[...], qkvb_ref[...],
               awb_ref[...], w2b_ref[...], b2b_ref[...])

    o_ref[...] = jnp.swapaxes(h.reshape(_J, f, 256)[:17], 0, 1)


def kernel(x, c1_l1w, c1_l1b, c1_qkvw, c1_aw, c1_l2w, c1_l2b, ln_g, ln_b,
           c2_l1w, c2_l1b, c2_qkvw, c2_aw, c2_l2w, c2_l2b):
    B, T, J, C = x.shape
    n = B * T
    g = n // _F
    xf = x.reshape(n, J, C)

    def pack(qkvw, aw):
        wp = qkvw.T[:, _PERM]
        aw_blk = jnp.tile(aw[0], _H)[:, None] * jnp.asarray(_AW_ONEHOT)
        return wp, aw_blk

    wp1, awb1 = pack(c1_qkvw, c1_aw)
    wp2, awb2 = pack(c2_qkvw, c2_aw)
    operands = (
        xf, jnp.asarray(_MADD), jnp.asarray(_E8),
        c1_l1w.T, c1_l1b[None], wp1, awb1, c1_l2w.T, c1_l2b[None],
        ln_g[None], ln_b[None],
        c2_l1w.T, c2_l1b[None], wp2, awb2, c2_l2w.T, c2_l2b[None],
    )

    def full_spec(a):
        return pl.BlockSpec(a.shape, lambda i, _r=len(a.shape): (0,) * _r)

    in_specs = [pl.BlockSpec((_F, J, C), lambda i: (i, 0, 0))]
    in_specs += [full_spec(a) for a in operands[1:]]

    out = pl.pallas_call(
        _fused,
        grid=(g,),
        in_specs=in_specs,
        out_specs=pl.BlockSpec((_F, J, C), lambda i: (i, 0, 0)),
        out_shape=jax.ShapeDtypeStruct((n, J, C), x.dtype),
        compiler_params=pltpu.CompilerParams(
            dimension_semantics=("parallel",)),
    )(*operands)
    return out.reshape(B, T, J, C)


# speedup vs baseline: 5.3673x; 1.0784x over previous
import numpy as np
import jax
import jax.numpy as jnp
from jax.experimental import pallas as pl
from jax.experimental.pallas import tpu as pltpu

_H = 8
_A = 32
_DH = 16
_J = 19
_F = 16

_SRC = [0, 0, 0, 1, 1, 2, 2, 3, 4, 4, 5, 5, 6, 7, 7, 8, 8, 8, 8, 9, 9, 10,
        11, 11, 12, 12, 13, 14, 14, 15, 15, 16]
_DST = [1, 4, 7, 0, 2, 1, 3, 2, 0, 5, 4, 6, 5, 0, 8, 7, 9, 11, 14, 8, 10, 9,
        8, 12, 11, 13, 12, 8, 15, 14, 16, 15]


def _edge_mask() -> np.ndarray:
    adj = np.zeros((17, 17), dtype=np.float32)
    adj[np.array(_SRC), np.array(_DST)] = 1.0
    a = adj + adj @ adj
    np.fill_diagonal(a, 0.0)
    src, dst = np.nonzero(a)
    src, dst = list(src), list(dst)
    for g in (17, 18):
        for i in range(17):
            src.append(i); dst.append(g)
            src.append(g); dst.append(i)
    madd = np.full((_J, _J), -1e30, dtype=np.float32)
    madd[np.array(src), np.array(dst)] = 0.0
    return madd


_MADD = _edge_mask()

_PERM = np.concatenate([
    np.array([h * 96 + a for h in range(_H) for a in range(_A)]),
    np.array([h * 96 + _A + a for h in range(_H) for a in range(_A)]),
    np.array([h * 96 + 2 * _A + d for h in range(_H) for d in range(_DH)]),
    np.array([h * 96 + 2 * _A + _DH + d for h in range(_H) for d in range(_DH)]),
])

_AW_ONEHOT = ((np.arange(_H * _DH)[None, :] // _DH)
              == (np.arange(_H * _A) // _A)[:, None]).astype(np.float32)


def _gconv(h, f, madd, w1, b1, wqkv, aw_e8, w2, b2):
    h1 = jnp.dot(h, w1, preferred_element_type=jnp.float32) + b1
    qkv = jnp.dot(h1, wqkv, preferred_element_type=jnp.float32)
    q = qkv[:, 0:256].reshape(_J, f, 256)
    k = qkv[:, 256:512].reshape(_J, f, 256)
    v1 = qkv[:, 512:640]
    v2 = qkv[:, 640:768].reshape(_J, f, 128)

    scores = []
    m128 = jnp.full((f, 128), -1e30, dtype=jnp.float32)
    for j in range(_J):
        sp = jax.nn.softplus(q + k[j][None])
        sj = jnp.dot(sp.reshape(_J * f, 256), aw_e8,
                     preferred_element_type=jnp.float32).reshape(_J, f, 128)
        sj = sj + madd[:, j:j + 1].reshape(_J, 1, 1)
        scores.append(sj)
        m128 = jnp.maximum(m128, jnp.max(sj, axis=0))
    den = jnp.zeros((_J, f, 128), dtype=jnp.float32)
    acc = jnp.zeros((_J, f, 128), dtype=jnp.float32)
    for j in range(_J):
        zj = jnp.exp(scores[j] - m128[None])
        den = den + zj
        acc = acc + zj * v2[j][None]
    out = v1 + acc.reshape(_J * f, 128) / (den.reshape(_J * f, 128) + 1e-10)
    return jnp.dot(out, w2, preferred_element_type=jnp.float32) + b2


def _fused(x_ref, madd_ref,
           w1a_ref, b1a_ref, qkva_ref, awa_ref, w2a_ref, b2a_ref,
           lng_ref, lnb_ref,
           w1b_ref, b1b_ref, qkvb_ref, awb_ref, w2b_ref, b2b_ref,
           o_ref):
    f = x_ref.shape[0]
    xt = jnp.swapaxes(x_ref[...], 0, 1)
    xm = jnp.mean(xt, axis=0, keepdims=True)
    h = jnp.concatenate(
        [xt, jnp.zeros((1, f, 256), jnp.float32), xm], axis=0)
    h = h.reshape(_J * f, 256)
    madd = madd_ref[...]

    h = _gconv(h, f, madd, w1a_ref[...], b1a_ref[...], qkva_ref[...],
               awa_ref[...], w2a_ref[...], b2a_ref[...])

    mu = jnp.mean(h, axis=-1, keepdims=True)
    var = jnp.mean((h - mu) ** 2, axis=-1, keepdims=True)
    h = (h - mu) * jax.lax.rsqrt(var + 1e-5) * lng_ref[...] + lnb_ref[...]

    h = _gconv(h, f, madd, w1b_ref[...], b1b_ref[...], qkvb_ref[...],
               awb_ref[...], w2b_ref[...], b2b_ref[...])

    o_ref[...] = jnp.swapaxes(h.reshape(_J, f, 256)[:17], 0, 1)


def kernel(x, c1_l1w, c1_l1b, c1_qkvw, c1_aw, c1_l2w, c1_l2b, ln_g, ln_b,
           c2_l1w, c2_l1b, c2_qkvw, c2_aw, c2_l2w, c2_l2b):
    B, T, J, C = x.shape
    n = B * T
    g = n // _F
    xf = x.reshape(n, J, C)

    def pack(qkvw, aw):
        wp = qkvw.T[:, _PERM]
        aw_e8 = jnp.tile(aw[0], _H)[:, None] * jnp.asarray(_AW_ONEHOT)
        return wp, aw_e8

    wp1, awb1 = pack(c1_qkvw, c1_aw)
    wp2, awb2 = pack(c2_qkvw, c2_aw)
    operands = (
        xf, jnp.asarray(_MADD),
        c1_l1w.T, c1_l1b[None], wp1, awb1, c1_l2w.T, c1_l2b[None],
        ln_g[None], ln_b[None],
        c2_l1w.T, c2_l1b[None], wp2, awb2, c2_l2w.T, c2_l2b[None],
    )

    def full_spec(a):
        return pl.BlockSpec(a.shape, lambda i, _r=len(a.shape): (0,) * _r)

    in_specs = [pl.BlockSpec((_F, J, C), lambda i: (i, 0, 0))]
    in_specs += [full_spec(a) for a in operands[1:]]

    out = pl.pallas_call(
        _fused,
        grid=(g,),
        in_specs=in_specs,
        out_specs=pl.BlockSpec((_F, J, C), lambda i: (i, 0, 0)),
        out_shape=jax.ShapeDtypeStruct((n, J, C), x.dtype),
        compiler_params=pltpu.CompilerParams(
            dimension_semantics=("parallel",)),
    )(*operands)
    return out.reshape(B, T, J, C)


# speedup vs baseline: 5.7092x; 1.0637x over previous
import numpy as np
import jax
import jax.numpy as jnp
from jax.experimental import pallas as pl
from jax.experimental.pallas import tpu as pltpu

_H = 8
_A = 32
_DH = 16
_J = 19
_F = 16

_SRC = [0, 0, 0, 1, 1, 2, 2, 3, 4, 4, 5, 5, 6, 7, 7, 8, 8, 8, 8, 9, 9, 10,
        11, 11, 12, 12, 13, 14, 14, 15, 15, 16]
_DST = [1, 4, 7, 0, 2, 1, 3, 2, 0, 5, 4, 6, 5, 0, 8, 7, 9, 11, 14, 8, 10, 9,
        8, 12, 11, 13, 12, 8, 15, 14, 16, 15]


def _edge_mask() -> np.ndarray:
    adj = np.zeros((17, 17), dtype=np.float32)
    adj[np.array(_SRC), np.array(_DST)] = 1.0
    a = adj + adj @ adj
    np.fill_diagonal(a, 0.0)
    src, dst = np.nonzero(a)
    src, dst = list(src), list(dst)
    for g in (17, 18):
        for i in range(17):
            src.append(i); dst.append(g)
            src.append(g); dst.append(i)
    madd = np.full((_J, _J), -1e30, dtype=np.float32)
    madd[np.array(src), np.array(dst)] = 0.0
    return madd


_MADD = _edge_mask()

_PERM = np.concatenate([
    np.array([h * 96 + a for h in range(_H) for a in range(_A)]),
    np.array([h * 96 + _A + a for h in range(_H) for a in range(_A)]),
    np.array([h * 96 + 2 * _A + d for h in range(_H) for d in range(_DH)]),
    np.array([h * 96 + 2 * _A + _DH + d for h in range(_H) for d in range(_DH)]),
])

_AW_ONEHOT = ((np.arange(_H * _DH)[None, :] // _DH)
              == (np.arange(_H * _A) // _A)[:, None]).astype(np.float32)


def _softplus(x):
    return jnp.maximum(x, 0.0) + jnp.log1p(jnp.exp(-jnp.abs(x)))


def _gconv(h, f, madd, w1, b1, wqkv, aw_e8, w2, b2):
    h1 = jnp.dot(h, w1, preferred_element_type=jnp.float32) + b1
    qkv = jnp.dot(h1, wqkv, preferred_element_type=jnp.float32)
    q = qkv[:, 0:256].reshape(_J, f, 256)
    k = qkv[:, 256:512].reshape(_J, f, 256)
    v1 = qkv[:, 512:640]
    v2 = qkv[:, 640:768].reshape(_J, f, 128)

    scores = []
    m128 = jnp.full((f, 128), -1e30, dtype=jnp.float32)
    for j in range(_J):
        sp = _softplus(q + k[j][None])
        sj = jnp.dot(sp.reshape(_J * f, 256), aw_e8,
                     preferred_element_type=jnp.float32).reshape(_J, f, 128)
        sj = sj + madd[:, j:j + 1].reshape(_J, 1, 1)
        scores.append(sj)
        m128 = jnp.maximum(m128, jnp.max(sj, axis=0))
    den = jnp.zeros((_J, f, 128), dtype=jnp.float32)
    acc = jnp.zeros((_J, f, 128), dtype=jnp.float32)
    for j in range(_J):
        zj = jnp.exp(scores[j] - m128[None])
        den = den + zj
        acc = acc + zj * v2[j][None]
    out = v1 + acc.reshape(_J * f, 128) / (den.reshape(_J * f, 128) + 1e-10)
    return jnp.dot(out, w2, preferred_element_type=jnp.float32) + b2


def _fused(x_ref, madd_ref,
           w1a_ref, b1a_ref, qkva_ref, awa_ref, w2a_ref, b2a_ref,
           lng_ref, lnb_ref,
           w1b_ref, b1b_ref, qkvb_ref, awb_ref, w2b_ref, b2b_ref,
           o_ref):
    f = x_ref.shape[0]
    xt = jnp.swapaxes(x_ref[...], 0, 1)
    xm = jnp.mean(xt, axis=0, keepdims=True)
    h = jnp.concatenate(
        [xt, jnp.zeros((1, f, 256), jnp.float32), xm], axis=0)
    h = h.reshape(_J * f, 256)
    madd = madd_ref[...]

    h = _gconv(h, f, madd, w1a_ref[...], b1a_ref[...], qkva_ref[...],
               awa_ref[...], w2a_ref[...], b2a_ref[...])

    mu = jnp.mean(h, axis=-1, keepdims=True)
    var = jnp.mean((h - mu) ** 2, axis=-1, keepdims=True)
    h = (h - mu) * jax.lax.rsqrt(var + 1e-5) * lng_ref[...] + lnb_ref[...]

    h = _gconv(h, f, madd, w1b_ref[...], b1b_ref[...], qkvb_ref[...],
               awb_ref[...], w2b_ref[...], b2b_ref[...])

    o_ref[...] = jnp.swapaxes(h.reshape(_J, f, 256)[:17], 0, 1)


def kernel(x, c1_l1w, c1_l1b, c1_qkvw, c1_aw, c1_l2w, c1_l2b, ln_g, ln_b,
           c2_l1w, c2_l1b, c2_qkvw, c2_aw, c2_l2w, c2_l2b):
    B, T, J, C = x.shape
    n = B * T
    g = n // _F
    xf = x.reshape(n, J, C)

    def pack(qkvw, aw):
        wp = qkvw.T[:, _PERM]
        aw_e8 = jnp.tile(aw[0], _H)[:, None] * jnp.asarray(_AW_ONEHOT)
        return wp, aw_e8

    wp1, awb1 = pack(c1_qkvw, c1_aw)
    wp2, awb2 = pack(c2_qkvw, c2_aw)
    operands = (
        xf, jnp.asarray(_MADD),
        c1_l1w.T, c1_l1b[None], wp1, awb1, c1_l2w.T, c1_l2b[None],
        ln_g[None], ln_b[None],
        c2_l1w.T, c2_l1b[None], wp2, awb2, c2_l2w.T, c2_l2b[None],
    )

    def full_spec(a):
        return pl.BlockSpec(a.shape, lambda i, _r=len(a.shape): (0,) * _r)

    in_specs = [pl.BlockSpec((_F, J, C), lambda i: (i, 0, 0))]
    in_specs += [full_spec(a) for a in operands[1:]]

    out = pl.pallas_call(
        _fused,
        grid=(g,),
        in_specs=in_specs,
        out_specs=pl.BlockSpec((_F, J, C), lambda i: (i, 0, 0)),
        out_shape=jax.ShapeDtypeStruct((n, J, C), x.dtype),
        compiler_params=pltpu.CompilerParams(
            dimension_semantics=("parallel",)),
    )(*operands)
    return out.reshape(B, T, J, C)


# speedup vs baseline: 5.7263x; 1.0030x over previous
import numpy as np
import jax
import jax.numpy as jnp
from jax.experimental import pallas as pl
from jax.experimental.pallas import tpu as pltpu

_H = 8
_A = 32
_DH = 16
_J = 19
_F = 16

_SRC = [0, 0, 0, 1, 1, 2, 2, 3, 4, 4, 5, 5, 6, 7, 7, 8, 8, 8, 8, 9, 9, 10,
        11, 11, 12, 12, 13, 14, 14, 15, 15, 16]
_DST = [1, 4, 7, 0, 2, 1, 3, 2, 0, 5, 4, 6, 5, 0, 8, 7, 9, 11, 14, 8, 10, 9,
        8, 12, 11, 13, 12, 8, 15, 14, 16, 15]


def _edge_mask() -> np.ndarray:
    adj = np.zeros((17, 17), dtype=np.float32)
    adj[np.array(_SRC), np.array(_DST)] = 1.0
    a = adj + adj @ adj
    np.fill_diagonal(a, 0.0)
    src, dst = np.nonzero(a)
    src, dst = list(src), list(dst)
    for g in (17, 18):
        for i in range(17):
            src.append(i); dst.append(g)
            src.append(g); dst.append(i)
    madd = np.full((_J, _J), -1e30, dtype=np.float32)
    madd[np.array(src), np.array(dst)] = 0.0
    return madd


_MADD = _edge_mask()

_AW_ONEHOT = ((np.arange(_H * _DH)[None, :] // _DH)
              == (np.arange(_H * _A) // _A)[:, None]).astype(np.float32)


def _pack_weights(qkvw_ref, l1w_ref, l2w_ref, aw_ref, oh_ref,
                  wp_ref, w1_ref, w2_ref, awe_ref):
    w = qkvw_ref[...].reshape(_H, 96, 128)
    packed = jnp.concatenate(
        [w[:, 0:32].reshape(_H * _A, 128),
         w[:, 32:64].reshape(_H * _A, 128),
         w[:, 64:80].reshape(_H * _DH, 128),
         w[:, 80:96].reshape(_H * _DH, 128)], axis=0)
    wp_ref[...] = jnp.swapaxes(packed, 0, 1)
    w1_ref[...] = jnp.swapaxes(l1w_ref[...], 0, 1)
    w2_ref[...] = jnp.swapaxes(l2w_ref[...], 0, 1)
    aw_col = aw_ref[...]
    awe_ref[...] = jnp.concatenate([aw_col] * _H, axis=0) * oh_ref[...]


def _softplus(x):
    return jnp.maximum(x, 0.0) + jnp.log1p(jnp.exp(-jnp.abs(x)))


def _gconv(h, f, madd, w1, b1, wqkv, aw_e8, w2, b2):
    h1 = jnp.dot(h, w1, preferred_element_type=jnp.float32) + b1
    qkv = jnp.dot(h1, wqkv, preferred_element_type=jnp.float32)
    q = qkv[:, 0:256].reshape(_J, f, 256)
    k = qkv[:, 256:512].reshape(_J, f, 256)
    v1 = qkv[:, 512:640]
    v2 = qkv[:, 640:768].reshape(_J, f, 128)

    scores = []
    m128 = jnp.full((f, 128), -1e30, dtype=jnp.float32)
    for j in range(_J):
        sp = _softplus(q + k[j][None])
        sj = jnp.dot(sp.reshape(_J * f, 256), aw_e8,
                     preferred_element_type=jnp.float32).reshape(_J, f, 128)
        sj = sj + madd[:, j:j + 1].reshape(_J, 1, 1)
        scores.append(sj)
        m128 = jnp.maximum(m128, jnp.max(sj, axis=0))
    den = jnp.zeros((_J, f, 128), dtype=jnp.float32)
    acc = jnp.zeros((_J, f, 128), dtype=jnp.float32)
    for j in range(_J):
        zj = jnp.exp(scores[j] - m128[None])
        den = den + zj
        acc = acc + zj * v2[j][None]
    out = v1 + acc.reshape(_J * f, 128) / (den.reshape(_J * f, 128) + 1e-10)
    return jnp.dot(out, w2, preferred_element_type=jnp.float32) + b2


def _fused(x_ref, madd_ref,
           w1a_ref, b1a_ref, qkva_ref, awa_ref, w2a_ref, b2a_ref,
           lng_ref, lnb_ref,
           w1b_ref, b1b_ref, qkvb_ref, awb_ref, w2b_ref, b2b_ref,
           o_ref):
    f = x_ref.shape[0]
    xt = jnp.swapaxes(x_ref[...], 0, 1)
    xm = jnp.mean(xt, axis=0, keepdims=True)
    h = jnp.concatenate(
        [xt, jnp.zeros((1, f, 256), jnp.float32), xm], axis=0)
    h = h.reshape(_J * f, 256)
    madd = madd_ref[...]

    h = _gconv(h, f, madd, w1a_ref[...], b1a_ref[...], qkva_ref[...],
               awa_ref[...], w2a_ref[...], b2a_ref[...])

    mu = jnp.mean(h, axis=-1, keepdims=True)
    var = jnp.mean((h - mu) ** 2, axis=-1, keepdims=True)
    h = (h - mu) * jax.lax.rsqrt(var + 1e-5) * lng_ref[...] + lnb_ref[...]

    h = _gconv(h, f, madd, w1b_ref[...], b1b_ref[...], qkvb_ref[...],
               awb_ref[...], w2b_ref[...], b2b_ref[...])

    o_ref[...] = jnp.swapaxes(h.reshape(_J, f, 256)[:17], 0, 1)


def kernel(x, c1_l1w, c1_l1b, c1_qkvw, c1_aw, c1_l2w, c1_l2b, ln_g, ln_b,
           c2_l1w, c2_l1b, c2_qkvw, c2_aw, c2_l2w, c2_l2b):
    B, T, J, C = x.shape
    n = B * T
    g = n // _F
    xf = x.reshape(n, J, C)

    onehot = jnp.asarray(_AW_ONEHOT)

    def pack(qkvw, l1w, l2w, aw):
        return pl.pallas_call(
            _pack_weights,
            out_shape=(jax.ShapeDtypeStruct((128, 768), x.dtype),
                       jax.ShapeDtypeStruct((256, 128), x.dtype),
                       jax.ShapeDtypeStruct((128, 256), x.dtype),
                       jax.ShapeDtypeStruct((_H * _A, 128), x.dtype)),
        )(qkvw, l1w, l2w, aw.reshape(_A, 1), onehot)

    wp1, w1a, w2a, awb1 = pack(c1_qkvw, c1_l1w, c1_l2w, c1_aw)
    wp2, w1b, w2b, awb2 = pack(c2_qkvw, c2_l1w, c2_l2w, c2_aw)
    operands = (
        xf, jnp.asarray(_MADD),
        w1a, c1_l1b[None], wp1, awb1, w2a, c1_l2b[None],
        ln_g[None], ln_b[None],
        w1b, c2_l1b[None], wp2, awb2, w2b, c2_l2b[None],
    )

    def full_spec(a):
        return pl.BlockSpec(a.shape, lambda i, _r=len(a.shape): (0,) * _r)

    in_specs = [pl.BlockSpec((_F, J, C), lambda i: (i, 0, 0))]
    in_specs += [full_spec(a) for a in operands[1:]]

    out = pl.pallas_call(
        _fused,
        grid=(g,),
        in_specs=in_specs,
        out_specs=pl.BlockSpec((_F, J, C), lambda i: (i, 0, 0)),
        out_shape=jax.ShapeDtypeStruct((n, J, C), x.dtype),
        compiler_params=pltpu.CompilerParams(
            dimension_semantics=("parallel",)),
    )(*operands)
    return out.reshape(B, T, J, C)


# speedup vs baseline: 6.1280x; 1.0701x over previous
import numpy as np
import jax
import jax.numpy as jnp
from jax.experimental import pallas as pl
from jax.experimental.pallas import tpu as pltpu

_H = 8
_A = 32
_DH = 16
_J = 19
_F = 16

_SRC = [0, 0, 0, 1, 1, 2, 2, 3, 4, 4, 5, 5, 6, 7, 7, 8, 8, 8, 8, 9, 9, 10,
        11, 11, 12, 12, 13, 14, 14, 15, 15, 16]
_DST = [1, 4, 7, 0, 2, 1, 3, 2, 0, 5, 4, 6, 5, 0, 8, 7, 9, 11, 14, 8, 10, 9,
        8, 12, 11, 13, 12, 8, 15, 14, 16, 15]


def _edge_mask() -> np.ndarray:
    adj = np.zeros((17, 17), dtype=np.float32)
    adj[np.array(_SRC), np.array(_DST)] = 1.0
    a = adj + adj @ adj
    np.fill_diagonal(a, 0.0)
    src, dst = np.nonzero(a)
    src, dst = list(src), list(dst)
    for g in (17, 18):
        for i in range(17):
            src.append(i); dst.append(g)
            src.append(g); dst.append(i)
    madd = np.full((_J, _J), -1e30, dtype=np.float32)
    madd[np.array(src), np.array(dst)] = 0.0
    return madd


_MADD = _edge_mask()

_AW_ONEHOT = ((np.arange(_H * _DH)[None, :] // _DH)
              == (np.arange(_H * _A) // _A)[:, None]).astype(np.float32)


def _pack_weights(qkvw_ref, l1w_ref, l2w_ref, aw_ref, oh_ref,
                  wp_ref, w1_ref, w2_ref, awe_ref):
    w = qkvw_ref[...].reshape(_H, 96, 128)
    packed = jnp.concatenate(
        [w[:, 0:32].reshape(_H * _A, 128),
         w[:, 32:64].reshape(_H * _A, 128),
         w[:, 64:80].reshape(_H * _DH, 128),
         w[:, 80:96].reshape(_H * _DH, 128)], axis=0)
    wp_ref[...] = jnp.swapaxes(packed, 0, 1)
    w1_ref[...] = jnp.swapaxes(l1w_ref[...], 0, 1)
    w2_ref[...] = jnp.swapaxes(l2w_ref[...], 0, 1)
    aw_col = aw_ref[...]
    awe_ref[...] = jnp.concatenate([aw_col] * _H, axis=0) * oh_ref[...]


def _softplus(x):
    return jnp.maximum(x, 0.0) + jnp.log1p(jnp.exp(-jnp.abs(x)))


def _gconv(h, f, madd, w1, b1, wqkv, aw_e8, w2, b2):
    h1 = jnp.dot(h, w1, preferred_element_type=jnp.float32) + b1
    qkv = jnp.dot(h1, wqkv, preferred_element_type=jnp.float32)
    q = qkv[:, 0:256].reshape(_J, f, 256)
    k = qkv[:, 256:512].reshape(_J, f, 256)
    v1 = qkv[:, 512:640]
    v2 = qkv[:, 640:768].reshape(_J, f, 128)

    scores = []
    m128 = jnp.full((f, 128), -1e30, dtype=jnp.float32)
    for j in range(_J):
        sp = _softplus(q + k[j][None])
        sj = jnp.dot(sp.reshape(_J * f, 256), aw_e8,
                     preferred_element_type=jnp.float32).reshape(_J, f, 128)
        sj = sj + madd[:, j:j + 1].reshape(_J, 1, 1)
        scores.append(sj)
        m128 = jnp.maximum(m128, jnp.max(sj, axis=0))
    den = jnp.zeros((_J, f, 128), dtype=jnp.float32)
    acc = jnp.zeros((_J, f, 128), dtype=jnp.float32)
    for j in range(_J):
        zj = jnp.exp(scores[j] - m128[None])
        den = den + zj
        acc = acc + zj * v2[j][None]
    out = v1 + acc.reshape(_J * f, 128) / (den.reshape(_J * f, 128) + 1e-10)
    return jnp.dot(out, w2, preferred_element_type=jnp.float32) + b2


def _fused(x_ref, madd_ref,
           w1a_ref, b1a_ref, qkva_ref, awa_ref, w2a_ref, b2a_ref,
           lng_ref, lnb_ref,
           w1b_ref, b1b_ref, qkvb_ref, awb_ref, w2b_ref, b2b_ref,
           o_ref):
    f = x_ref.shape[0]
    xt = jnp.swapaxes(x_ref[...].reshape(f, 17, 256), 0, 1)
    xm = jnp.mean(xt, axis=0, keepdims=True)
    h = jnp.concatenate(
        [xt, jnp.zeros((1, f, 256), jnp.float32), xm], axis=0)
    h = h.reshape(_J * f, 256)
    madd = madd_ref[...]

    h = _gconv(h, f, madd, w1a_ref[...], b1a_ref[...], qkva_ref[...],
               awa_ref[...], w2a_ref[...], b2a_ref[...])

    mu = jnp.mean(h, axis=-1, keepdims=True)
    var = jnp.mean((h - mu) ** 2, axis=-1, keepdims=True)
    h = (h - mu) * jax.lax.rsqrt(var + 1e-5) * lng_ref[...] + lnb_ref[...]

    h = _gconv(h, f, madd, w1b_ref[...], b1b_ref[...], qkvb_ref[...],
               awb_ref[...], w2b_ref[...], b2b_ref[...])

    o_ref[...] = jnp.swapaxes(h.reshape(_J, f, 256)[:17], 0, 1
                              ).reshape(f, 1, 17, 256)


def kernel(x, c1_l1w, c1_l1b, c1_qkvw, c1_aw, c1_l2w, c1_l2b, ln_g, ln_b,
           c2_l1w, c2_l1b, c2_qkvw, c2_aw, c2_l2w, c2_l2b):
    B, T, J, C = x.shape

    onehot = jnp.asarray(_AW_ONEHOT)

    def pack(qkvw, l1w, l2w, aw):
        return pl.pallas_call(
            _pack_weights,
            out_shape=(jax.ShapeDtypeStruct((128, 768), x.dtype),
                       jax.ShapeDtypeStruct((256, 128), x.dtype),
                       jax.ShapeDtypeStruct((128, 256), x.dtype),
                       jax.ShapeDtypeStruct((_H * _A, 128), x.dtype)),
        )(qkvw, l1w, l2w, aw.reshape(_A, 1), onehot)

    wp1, w1a, w2a, awb1 = pack(c1_qkvw, c1_l1w, c1_l2w, c1_aw)
    wp2, w1b, w2b, awb2 = pack(c2_qkvw, c2_l1w, c2_l2w, c2_aw)
    operands = (
        x, jnp.asarray(_MADD),
        w1a, c1_l1b[None], wp1, awb1, w2a, c1_l2b[None],
        ln_g[None], ln_b[None],
        w1b, c2_l1b[None], wp2, awb2, w2b, c2_l2b[None],
    )

    def full_spec(a):
        return pl.BlockSpec(a.shape, lambda i, _r=len(a.shape): (0,) * _r)

    in_specs = [pl.BlockSpec((B, 1, J, C), lambda i: (0, i, 0, 0))]
    in_specs += [full_spec(a) for a in operands[1:]]

    out = pl.pallas_call(
        _fused,
        grid=(T,),
        in_specs=in_specs,
        out_specs=pl.BlockSpec((B, 1, J, C), lambda i: (0, i, 0, 0)),
        out_shape=jax.ShapeDtypeStruct((B, T, J, C), x.dtype),
        compiler_params=pltpu.CompilerParams(
            dimension_semantics=("parallel",)),
    )(*operands)
    return out


# speedup vs baseline: 12.0856x; 1.9722x over previous
import numpy as np
import jax
import jax.numpy as jnp
from jax.experimental import pallas as pl
from jax.experimental.pallas import tpu as pltpu

_H = 8
_A = 32
_DH = 16
_J = 19
_F = 16

_SRC = [0, 0, 0, 1, 1, 2, 2, 3, 4, 4, 5, 5, 6, 7, 7, 8, 8, 8, 8, 9, 9, 10,
        11, 11, 12, 12, 13, 14, 14, 15, 15, 16]
_DST = [1, 4, 7, 0, 2, 1, 3, 2, 0, 5, 4, 6, 5, 0, 8, 7, 9, 11, 14, 8, 10, 9,
        8, 12, 11, 13, 12, 8, 15, 14, 16, 15]


def _edge_preds() -> list:
    adj = np.zeros((17, 17), dtype=np.float32)
    adj[np.array(_SRC), np.array(_DST)] = 1.0
    a = adj + adj @ adj
    np.fill_diagonal(a, 0.0)
    src, dst = np.nonzero(a)
    src, dst = list(src), list(dst)
    for g in (17, 18):
        for i in range(17):
            src.append(i); dst.append(g)
            src.append(g); dst.append(i)
    preds = [[] for _ in range(_J)]
    for i, j in zip(src, dst):
        preds[j].append(int(i))
    return preds


_PREDS = _edge_preds()

_AW_ONEHOT = ((np.arange(_H * _DH)[None, :] // _DH)
              == (np.arange(_H * _A) // _A)[:, None]).astype(np.float32)


def _pack_weights(qkvw_ref, l1w_ref, l2w_ref, aw_ref, oh_ref,
                  wp_ref, w1_ref, w2_ref, awe_ref):
    w = qkvw_ref[...].reshape(_H, 96, 128)
    packed = jnp.concatenate(
        [w[:, 0:32].reshape(_H * _A, 128),
         w[:, 32:64].reshape(_H * _A, 128),
         w[:, 64:80].reshape(_H * _DH, 128),
         w[:, 80:96].reshape(_H * _DH, 128)], axis=0)
    wp_ref[...] = jnp.swapaxes(packed, 0, 1)
    w1_ref[...] = jnp.swapaxes(l1w_ref[...], 0, 1)
    w2_ref[...] = jnp.swapaxes(l2w_ref[...], 0, 1)
    aw_col = aw_ref[...]
    awe_ref[...] = jnp.concatenate([aw_col] * _H, axis=0) * oh_ref[...]


def _softplus(x):
    return jnp.maximum(x, 0.0) + jnp.log1p(jnp.exp(-jnp.abs(x)))


def _gconv(h, f, w1, b1, wqkv, aw_e8, w2, b2):
    h1 = jnp.dot(h, w1, preferred_element_type=jnp.float32) + b1
    qkv = jnp.dot(h1, wqkv, preferred_element_type=jnp.float32)
    q = qkv[:, 0:256].reshape(_J, f, 256)
    k = qkv[:, 256:512].reshape(_J, f, 256)
    v1 = qkv[:, 512:640]
    v2 = qkv[:, 640:768].reshape(_J, f, 128)

    groups = []
    m128 = jnp.full((f, 128), -1e30, dtype=jnp.float32)
    for j in range(_J):
        pred = _PREDS[j]
        qg = jnp.concatenate([q[i:i + 1] for i in pred], axis=0)
        sp = _softplus(qg + k[j:j + 1])
        sg = jnp.dot(sp.reshape(len(pred) * f, 256), aw_e8,
                     preferred_element_type=jnp.float32
                     ).reshape(len(pred), f, 128)
        groups.append(sg)
        m128 = jnp.maximum(m128, jnp.max(sg, axis=0))
    den = [None] * _J
    acc = [None] * _J
    for j in range(_J):
        zg = jnp.exp(groups[j] - m128[None])
        v2j = v2[j]
        for idx, i in enumerate(_PREDS[j]):
            zi = zg[idx]
            ai = zi * v2j
            den[i] = zi if den[i] is None else den[i] + zi
            acc[i] = ai if acc[i] is None else acc[i] + ai
    den_all = jnp.concatenate([d[None] for d in den], axis=0)
    acc_all = jnp.concatenate([a[None] for a in acc], axis=0)
    out = v1 + acc_all.reshape(_J * f, 128) / (
        den_all.reshape(_J * f, 128) + 1e-10)
    return jnp.dot(out, w2, preferred_element_type=jnp.float32) + b2


def _fused(x_ref,
           w1a_ref, b1a_ref, qkva_ref, awa_ref, w2a_ref, b2a_ref,
           lng_ref, lnb_ref,
           w1b_ref, b1b_ref, qkvb_ref, awb_ref, w2b_ref, b2b_ref,
           o_ref):
    f = x_ref.shape[0]
    xt = jnp.swapaxes(x_ref[...].reshape(f, 17, 256), 0, 1)
    xm = jnp.mean(xt, axis=0, keepdims=True)
    h = jnp.concatenate(
        [xt, jnp.zeros((1, f, 256), jnp.float32), xm], axis=0)
    h = h.reshape(_J * f, 256)

    h = _gconv(h, f, w1a_ref[...], b1a_ref[...], qkva_ref[...],
               awa_ref[...], w2a_ref[...], b2a_ref[...])

    mu = jnp.mean(h, axis=-1, keepdims=True)
    var = jnp.mean((h - mu) ** 2, axis=-1, keepdims=True)
    h = (h - mu) * jax.lax.rsqrt(var + 1e-5) * lng_ref[...] + lnb_ref[...]

    h = _gconv(h, f, w1b_ref[...], b1b_ref[...], qkvb_ref[...],
               awb_ref[...], w2b_ref[...], b2b_ref[...])

    o_ref[...] = jnp.swapaxes(h.reshape(_J, f, 256)[:17], 0, 1
                              ).reshape(f, 1, 17, 256)


def kernel(x, c1_l1w, c1_l1b, c1_qkvw, c1_aw, c1_l2w, c1_l2b, ln_g, ln_b,
           c2_l1w, c2_l1b, c2_qkvw, c2_aw, c2_l2w, c2_l2b):
    B, T, J, C = x.shape

    onehot = jnp.asarray(_AW_ONEHOT)

    def pack(qkvw, l1w, l2w, aw):
        return pl.pallas_call(
            _pack_weights,
            out_shape=(jax.ShapeDtypeStruct((128, 768), x.dtype),
                       jax.ShapeDtypeStruct((256, 128), x.dtype),
                       jax.ShapeDtypeStruct((128, 256), x.dtype),
                       jax.ShapeDtypeStruct((_H * _A, 128), x.dtype)),
        )(qkvw, l1w, l2w, aw.reshape(_A, 1), onehot)

    wp1, w1a, w2a, awb1 = pack(c1_qkvw, c1_l1w, c1_l2w, c1_aw)
    wp2, w1b, w2b, awb2 = pack(c2_qkvw, c2_l1w, c2_l2w, c2_aw)
    operands = (
        x,
        w1a, c1_l1b[None], wp1, awb1, w2a, c1_l2b[None],
        ln_g[None], ln_b[None],
        w1b, c2_l1b[None], wp2, awb2, w2b, c2_l2b[None],
    )

    def full_spec(a):
        return pl.BlockSpec(a.shape, lambda i, _r=len(a.shape): (0,) * _r)

    in_specs = [pl.BlockSpec((B, 1, J, C), lambda i: (0, i, 0, 0))]
    in_specs += [full_spec(a) for a in operands[1:]]

    out = pl.pallas_call(
        _fused,
        grid=(T,),
        in_specs=in_specs,
        out_specs=pl.BlockSpec((B, 1, J, C), lambda i: (0, i, 0, 0)),
        out_shape=jax.ShapeDtypeStruct((B, T, J, C), x.dtype),
        compiler_params=pltpu.CompilerParams(
            dimension_semantics=("parallel",)),
    )(*operands)
    return out


# speedup vs baseline: 14.3658x; 1.1887x over previous
import numpy as np
import jax
import jax.numpy as jnp
from jax.experimental import pallas as pl
from jax.experimental.pallas import tpu as pltpu

_H = 8
_A = 32
_DH = 16
_J = 19
_TB = 3

_SRC = [0, 0, 0, 1, 1, 2, 2, 3, 4, 4, 5, 5, 6, 7, 7, 8, 8, 8, 8, 9, 9, 10,
        11, 11, 12, 12, 13, 14, 14, 15, 15, 16]
_DST = [1, 4, 7, 0, 2, 1, 3, 2, 0, 5, 4, 6, 5, 0, 8, 7, 9, 11, 14, 8, 10, 9,
        8, 12, 11, 13, 12, 8, 15, 14, 16, 15]


def _edge_preds() -> list:
    adj = np.zeros((17, 17), dtype=np.float32)
    adj[np.array(_SRC), np.array(_DST)] = 1.0
    a = adj + adj @ adj
    np.fill_diagonal(a, 0.0)
    src, dst = np.nonzero(a)
    src, dst = list(src), list(dst)
    for g in (17, 18):
        for i in range(17):
            src.append(i); dst.append(g)
            src.append(g); dst.append(i)
    preds = [[] for _ in range(_J)]
    for i, j in zip(src, dst):
        preds[j].append(int(i))
    return preds


_PREDS = _edge_preds()

_AW_ONEHOT = ((np.arange(_H * _DH)[None, :] // _DH)
              == (np.arange(_H * _A) // _A)[:, None]).astype(np.float32)


def _pack_weights(qkvw_ref, l1w_ref, l2w_ref, aw_ref, oh_ref,
                  wp_ref, w1_ref, w2_ref, awe_ref):
    w = qkvw_ref[...].reshape(_H, 96, 128)
    packed = jnp.concatenate(
        [w[:, 0:32].reshape(_H * _A, 128),
         w[:, 32:64].reshape(_H * _A, 128),
         w[:, 64:80].reshape(_H * _DH, 128),
         w[:, 80:96].reshape(_H * _DH, 128)], axis=0)
    wp_ref[...] = jnp.swapaxes(packed, 0, 1)
    w1_ref[...] = jnp.swapaxes(l1w_ref[...], 0, 1)
    w2_ref[...] = jnp.swapaxes(l2w_ref[...], 0, 1)
    aw_col = aw_ref[...]
    awe_ref[...] = jnp.concatenate([aw_col] * _H, axis=0) * oh_ref[...]


def _softplus(x):
    return jnp.maximum(x, 0.0) + jnp.log1p(jnp.exp(-jnp.abs(x)))


def _gconv(h, f, w1, b1, wqkv, aw_e8, w2, b2):
    h1 = jnp.dot(h, w1, preferred_element_type=jnp.float32) + b1
    qkv = jnp.dot(h1, wqkv, preferred_element_type=jnp.float32)
    q = qkv[:, 0:256].reshape(_J, f, 256)
    k = qkv[:, 256:512].reshape(_J, f, 256)
    v1 = qkv[:, 512:640]
    v2 = qkv[:, 640:768].reshape(_J, f, 128)

    groups = []
    m128 = jnp.full((f, 128), -1e30, dtype=jnp.float32)
    for j in range(_J):
        pred = _PREDS[j]
        qg = jnp.concatenate([q[i:i + 1] for i in pred], axis=0)
        sp = _softplus(qg + k[j:j + 1])
        sg = jnp.dot(sp.reshape(len(pred) * f, 256), aw_e8,
                     preferred_element_type=jnp.float32
                     ).reshape(len(pred), f, 128)
        groups.append(sg)
        m128 = jnp.maximum(m128, jnp.max(sg, axis=0))
    den = [None] * _J
    acc = [None] * _J
    for j in range(_J):
        zg = jnp.exp(groups[j] - m128[None])
        v2j = v2[j]
        for idx, i in enumerate(_PREDS[j]):
            zi = zg[idx]
            ai = zi * v2j
            den[i] = zi if den[i] is None else den[i] + zi
            acc[i] = ai if acc[i] is None else acc[i] + ai
    den_all = jnp.concatenate([d[None] for d in den], axis=0)
    acc_all = jnp.concatenate([a[None] for a in acc], axis=0)
    out = v1 + acc_all.reshape(_J * f, 128) / (
        den_all.reshape(_J * f, 128) + 1e-10)
    return jnp.dot(out, w2, preferred_element_type=jnp.float32) + b2


def _fused(x_ref,
           w1a_ref, b1a_ref, qkva_ref, awa_ref, w2a_ref, b2a_ref,
           lng_ref, lnb_ref,
           w1b_ref, b1b_ref, qkvb_ref, awb_ref, w2b_ref, b2b_ref,
           o_ref):
    f = x_ref.shape[0] * x_ref.shape[1]
    xt = jnp.swapaxes(x_ref[...].reshape(f, 17, 256), 0, 1)
    xm = jnp.mean(xt, axis=0, keepdims=True)
    h = jnp.concatenate(
        [xt, jnp.zeros((1, f, 256), jnp.float32), xm], axis=0)
    h = h.reshape(_J * f, 256)

    h = _gconv(h, f, w1a_ref[...], b1a_ref[...], qkva_ref[...],
               awa_ref[...], w2a_ref[...], b2a_ref[...])

    mu = jnp.mean(h, axis=-1, keepdims=True)
    var = jnp.mean((h - mu) ** 2, axis=-1, keepdims=True)
    h = (h - mu) * jax.lax.rsqrt(var + 1e-5) * lng_ref[...] + lnb_ref[...]

    h = _gconv(h, f, w1b_ref[...], b1b_ref[...], qkvb_ref[...],
               awb_ref[...], w2b_ref[...], b2b_ref[...])

    o_ref[...] = jnp.swapaxes(h.reshape(_J, f, 256)[:17], 0, 1
                              ).reshape(f // _TB, _TB, 17, 256)


def kernel(x, c1_l1w, c1_l1b, c1_qkvw, c1_aw, c1_l2w, c1_l2b, ln_g, ln_b,
           c2_l1w, c2_l1b, c2_qkvw, c2_aw, c2_l2w, c2_l2b):
    B, T, J, C = x.shape

    onehot = jnp.asarray(_AW_ONEHOT)

    def pack(qkvw, l1w, l2w, aw):
        return pl.pallas_call(
            _pack_weights,
            out_shape=(jax.ShapeDtypeStruct((128, 768), x.dtype),
                       jax.ShapeDtypeStruct((256, 128), x.dtype),
                       jax.ShapeDtypeStruct((128, 256), x.dtype),
                       jax.ShapeDtypeStruct((_H * _A, 128), x.dtype)),
        )(qkvw, l1w, l2w, aw.reshape(_A, 1), onehot)

    wp1, w1a, w2a, awb1 = pack(c1_qkvw, c1_l1w, c1_l2w, c1_aw)
    wp2, w1b, w2b, awb2 = pack(c2_qkvw, c2_l1w, c2_l2w, c2_aw)
    operands = (
        x,
        w1a, c1_l1b[None], wp1, awb1, w2a, c1_l2b[None],
        ln_g[None], ln_b[None],
        w1b, c2_l1b[None], wp2, awb2, w2b, c2_l2b[None],
    )

    def full_spec(a):
        return pl.BlockSpec(a.shape, lambda i, _r=len(a.shape): (0,) * _r)

    in_specs = [pl.BlockSpec((B, _TB, J, C), lambda i: (0, i, 0, 0))]
    in_specs += [full_spec(a) for a in operands[1:]]

    out = pl.pallas_call(
        _fused,
        grid=(T // _TB,),
        in_specs=in_specs,
        out_specs=pl.BlockSpec((B, _TB, J, C), lambda i: (0, i, 0, 0)),
        out_shape=jax.ShapeDtypeStruct((B, T, J, C), x.dtype),
        compiler_params=pltpu.CompilerParams(
            dimension_semantics=("parallel",)),
    )(*operands)
    return out


# speedup vs baseline: 14.8553x; 1.0341x over previous
import numpy as np
import jax
import jax.numpy as jnp
from jax.experimental import pallas as pl
from jax.experimental.pallas import tpu as pltpu

_H = 8
_A = 32
_DH = 16
_J = 19
_TB = 9

_SRC = [0, 0, 0, 1, 1, 2, 2, 3, 4, 4, 5, 5, 6, 7, 7, 8, 8, 8, 8, 9, 9, 10,
        11, 11, 12, 12, 13, 14, 14, 15, 15, 16]
_DST = [1, 4, 7, 0, 2, 1, 3, 2, 0, 5, 4, 6, 5, 0, 8, 7, 9, 11, 14, 8, 10, 9,
        8, 12, 11, 13, 12, 8, 15, 14, 16, 15]


def _edge_preds() -> list:
    adj = np.zeros((17, 17), dtype=np.float32)
    adj[np.array(_SRC), np.array(_DST)] = 1.0
    a = adj + adj @ adj
    np.fill_diagonal(a, 0.0)
    src, dst = np.nonzero(a)
    src, dst = list(src), list(dst)
    for g in (17, 18):
        for i in range(17):
            src.append(i); dst.append(g)
            src.append(g); dst.append(i)
    preds = [[] for _ in range(_J)]
    for i, j in zip(src, dst):
        preds[j].append(int(i))
    return preds


_PREDS = _edge_preds()

_AW_ONEHOT = ((np.arange(_H * _DH)[None, :] // _DH)
              == (np.arange(_H * _A) // _A)[:, None]).astype(np.float32)


def _pack_weights(qkvw_ref, l1w_ref, l2w_ref, aw_ref, oh_ref,
                  wp_ref, w1_ref, w2_ref, awe_ref):
    w = qkvw_ref[...].reshape(_H, 96, 128)
    packed = jnp.concatenate(
        [w[:, 0:32].reshape(_H * _A, 128),
         w[:, 32:64].reshape(_H * _A, 128),
         w[:, 64:80].reshape(_H * _DH, 128),
         w[:, 80:96].reshape(_H * _DH, 128)], axis=0)
    wp_ref[...] = jnp.swapaxes(packed, 0, 1)
    w1_ref[...] = jnp.swapaxes(l1w_ref[...], 0, 1)
    w2_ref[...] = jnp.swapaxes(l2w_ref[...], 0, 1)
    aw_col = aw_ref[...]
    awe_ref[...] = jnp.concatenate([aw_col] * _H, axis=0) * oh_ref[...]


def _softplus(x):
    return jnp.maximum(x, 0.0) + jnp.log1p(jnp.exp(-jnp.abs(x)))


def _gconv(h, f, w1, b1, wqkv, aw_e8, w2, b2):
    h1 = jnp.dot(h, w1, preferred_element_type=jnp.float32) + b1
    qkv = jnp.dot(h1, wqkv, preferred_element_type=jnp.float32)
    q = qkv[:, 0:256].reshape(_J, f, 256)
    k = qkv[:, 256:512].reshape(_J, f, 256)
    v1 = qkv[:, 512:640]
    v2 = qkv[:, 640:768].reshape(_J, f, 128)

    groups = []
    m128 = jnp.full((f, 128), -1e30, dtype=jnp.float32)
    for j in range(_J):
        pred = _PREDS[j]
        qg = jnp.concatenate([q[i:i + 1] for i in pred], axis=0)
        sp = _softplus(qg + k[j:j + 1])
        sg = jnp.dot(sp.reshape(len(pred) * f, 256), aw_e8,
                     preferred_element_type=jnp.float32
                     ).reshape(len(pred), f, 128)
        groups.append(sg)
        m128 = jnp.maximum(m128, jnp.max(sg, axis=0))
    den = [None] * _J
    acc = [None] * _J
    for j in range(_J):
        zg = jnp.exp(groups[j] - m128[None])
        v2j = v2[j]
        for idx, i in enumerate(_PREDS[j]):
            zi = zg[idx]
            ai = zi * v2j
            den[i] = zi if den[i] is None else den[i] + zi
            acc[i] = ai if acc[i] is None else acc[i] + ai
    den_all = jnp.concatenate([d[None] for d in den], axis=0)
    acc_all = jnp.concatenate([a[None] for a in acc], axis=0)
    out = v1 + acc_all.reshape(_J * f, 128) / (
        den_all.reshape(_J * f, 128) + 1e-10)
    return jnp.dot(out, w2, preferred_element_type=jnp.float32) + b2


def _fused(x_ref,
           w1a_ref, b1a_ref, qkva_ref, awa_ref, w2a_ref, b2a_ref,
           lng_ref, lnb_ref,
           w1b_ref, b1b_ref, qkvb_ref, awb_ref, w2b_ref, b2b_ref,
           o_ref):
    f = x_ref.shape[0] * x_ref.shape[1]
    xt = jnp.swapaxes(x_ref[...].reshape(f, 17, 256), 0, 1)
    xm = jnp.mean(xt, axis=0, keepdims=True)
    h = jnp.concatenate(
        [xt, jnp.zeros((1, f, 256), jnp.float32), xm], axis=0)
    h = h.reshape(_J * f, 256)

    h = _gconv(h, f, w1a_ref[...], b1a_ref[...], qkva_ref[...],
               awa_ref[...], w2a_ref[...], b2a_ref[...])

    mu = jnp.mean(h, axis=-1, keepdims=True)
    var = jnp.mean((h - mu) ** 2, axis=-1, keepdims=True)
    h = (h - mu) * jax.lax.rsqrt(var + 1e-5) * lng_ref[...] + lnb_ref[...]

    h = _gconv(h, f, w1b_ref[...], b1b_ref[...], qkvb_ref[...],
               awb_ref[...], w2b_ref[...], b2b_ref[...])

    o_ref[...] = jnp.swapaxes(h.reshape(_J, f, 256)[:17], 0, 1
                              ).reshape(f // _TB, _TB, 17, 256)


def kernel(x, c1_l1w, c1_l1b, c1_qkvw, c1_aw, c1_l2w, c1_l2b, ln_g, ln_b,
           c2_l1w, c2_l1b, c2_qkvw, c2_aw, c2_l2w, c2_l2b):
    B, T, J, C = x.shape

    onehot = jnp.asarray(_AW_ONEHOT)

    def pack(qkvw, l1w, l2w, aw):
        return pl.pallas_call(
            _pack_weights,
            out_shape=(jax.ShapeDtypeStruct((128, 768), x.dtype),
                       jax.ShapeDtypeStruct((256, 128), x.dtype),
                       jax.ShapeDtypeStruct((128, 256), x.dtype),
                       jax.ShapeDtypeStruct((_H * _A, 128), x.dtype)),
        )(qkvw, l1w, l2w, aw.reshape(_A, 1), onehot)

    wp1, w1a, w2a, awb1 = pack(c1_qkvw, c1_l1w, c1_l2w, c1_aw)
    wp2, w1b, w2b, awb2 = pack(c2_qkvw, c2_l1w, c2_l2w, c2_aw)
    operands = (
        x,
        w1a, c1_l1b[None], wp1, awb1, w2a, c1_l2b[None],
        ln_g[None], ln_b[None],
        w1b, c2_l1b[None], wp2, awb2, w2b, c2_l2b[None],
    )

    def full_spec(a):
        return pl.BlockSpec(a.shape, lambda i, _r=len(a.shape): (0,) * _r)

    in_specs = [pl.BlockSpec((B, _TB, J, C), lambda i: (0, i, 0, 0))]
    in_specs += [full_spec(a) for a in operands[1:]]

    out = pl.pallas_call(
        _fused,
        grid=(T // _TB,),
        in_specs=in_specs,
        out_specs=pl.BlockSpec((B, _TB, J, C), lambda i: (0, i, 0, 0)),
        out_shape=jax.ShapeDtypeStruct((B, T, J, C), x.dtype),
        compiler_params=pltpu.CompilerParams(
            dimension_semantics=("parallel",)),
    )(*operands)
    return out
